# Initial kernel scaffold; baseline (speedup 1.0000x reference)
import jax
import jax.numpy as jnp
from jax.experimental import pallas as pl


def kernel(x, batch, params):
    raise NotImplementedError("write your pallas kernel here")


# speedup vs baseline: 5.9853x; 5.9853x over previous
import functools
import numpy as np
import jax
import jax.numpy as jnp
from jax import lax
from jax.experimental import pallas as pl

N = 8192
NB = 8
K = 5
BNS = float(1.0 / np.sqrt(1.0 + 1e-5))

_NN = (((1,), (0,)), ((), ()))
_NT = (((1,), (1,)), ((), ()))
_TN = (((0,), (0,)), ((), ()))


def _mm(a, b):
    return lax.dot_general(a, b, _NN, preferred_element_type=jnp.float32)


def _mmt(a, b):
    return lax.dot_general(a, b, _NT, preferred_element_type=jnp.float32)


def _mtn(a, b):
    return lax.dot_general(a, b, _TN, preferred_element_type=jnp.float32)


def _ge_body(x_ref, b_ref, o_ref):
    x = x_ref[...]
    b = b_ref[...]
    oneh = (b == lax.broadcasted_iota(jnp.int32, (1, NB), 1)).astype(jnp.float32)
    ones_col = jnp.ones((N, 1), jnp.float32)
    cnt = _mtn(oneh, ones_col)
    ssum = _mtn(oneh, x)
    mean = ssum / jnp.maximum(cnt, 1.0)
    mns, mxs = [], []
    for bb in range(NB):
        m = b == bb
        mns.append(jnp.min(jnp.where(m, x, jnp.inf), axis=0, keepdims=True))
        mxs.append(jnp.max(jnp.where(m, x, -jnp.inf), axis=0, keepdims=True))
    mn = jnp.concatenate(mns, axis=0)
    mx = jnp.concatenate(mxs, axis=0)
    nonempty = cnt > 0.0
    mn = jnp.where(nonempty, mn, 0.0)
    mx = jnp.where(nonempty, mx, 0.0)
    mmm = jnp.concatenate([mean, mn, mx], axis=1)
    o_ref[...] = jnp.concatenate([_mm(oneh, mmm), x], axis=1)


def _global_exchange_call(x, bcol):
    return pl.pallas_call(
        _ge_body,
        out_shape=jax.ShapeDtypeStruct((N, 80), jnp.float32),
    )(x, bcol)


def _dense_body2(h_ref, w1_ref, bb1_ref, g1_ref, w2_ref, bb2_ref, g2_ref,
                 w3_ref, b3_ref, sw_ref, sb_ref, hw_ref, hb_ref,
                 feat_ref, s_ref, hm_ref):
    h = h_ref[...]
    bb1 = bb1_ref[...]
    x1 = jax.nn.relu(_mm(h, w1_ref[...]) + bb1[0:1, :])
    x1 = x1 * (g1_ref[...] * BNS) + bb1[1:2, :]
    bb2 = bb2_ref[...]
    x2 = jax.nn.relu(_mm(x1, w2_ref[...]) + bb2[0:1, :])
    x2 = x2 * (g2_ref[...] * BNS) + bb2[1:2, :]
    feat = _mm(x2, w3_ref[...]) + b3_ref[...]
    feat_ref[...] = feat
    s_ref[...] = _mm(feat, sw_ref[...]) + sb_ref[...]
    hm_ref[...] = _mm(feat, hw_ref[...]) + hb_ref[...]


def _dense_call(h, p, i):
    w1 = p[f'b{i}_W1'].T
    bb1 = jnp.stack([p[f'b{i}_b1'], p[f'b{i}_be1']])
    g1 = p[f'b{i}_g1'][None, :]
    w2 = p[f'b{i}_W2'].T
    bb2 = jnp.stack([p[f'b{i}_b2'], p[f'b{i}_be2']])
    g2 = p[f'b{i}_g2'][None, :]
    w3 = p[f'b{i}_W3'].T
    b3 = p[f'b{i}_b3'][None, :]
    sw = p[f'b{i}_sW'].T
    sb = p[f'b{i}_sb'][None, :]
    hw = p[f'b{i}_hW'].T
    hb = p[f'b{i}_hb'][None, :]
    return pl.pallas_call(
        _dense_body2,
        out_shape=[
            jax.ShapeDtypeStruct((N, 64), jnp.float32),
            jax.ShapeDtypeStruct((N, 16), jnp.float32),
            jax.ShapeDtypeStruct((N, 64), jnp.float32),
        ],
    )(h, w1, bb1, g1, w2, bb2, g2, w3, b3, sw, sb, hw, hb)


ROWT = 256
NEG_BIG = -1e9


def _gravnet_body2(sr_ref, br_ref, feat_ref, s_ref, ball_ref, h_ref,
                   ow_ref, obp_ref, pg_ref, o_ref):
    s_r = sr_ref[...]
    s_a = s_ref[...]
    sq_r = jnp.sum(s_r * s_r, axis=1, keepdims=True)
    sq_a = jnp.sum(s_a * s_a, axis=1, keepdims=True)
    aug_r = jnp.concatenate([s_r * -2.0, jnp.ones((ROWT, 1), jnp.float32)],
                            axis=1)
    aug_a = jnp.concatenate([s_a, sq_a], axis=1)
    d2 = sq_r + _mmt(aug_r, aug_a)
    d2 = jnp.maximum(d2, 0.0)
    same = br_ref[...] == ball_ref[...]
    cur = jnp.where(same, -d2, NEG_BIG)
    cols = lax.broadcasted_iota(jnp.int32, (ROWT, N), 1)
    h_a = h_ref[...]
    msum = jnp.zeros((ROWT, 64), jnp.float32)
    mmax = jnp.full((ROWT, 64), -jnp.inf, jnp.float32)
    for _ in range(K):
        mval = jnp.max(cur, axis=1, keepdims=True)
        eq = cur == mval
        amax = jnp.min(jnp.where(eq, cols, jnp.int32(1 << 30)),
                       axis=1, keepdims=True)
        sel = cols == amax
        oneh = sel.astype(jnp.float32)
        g = _mm(oneh, h_a)
        msg = g * jnp.exp(10.0 * mval)
        msum = msum + msg
        mmax = jnp.maximum(mmax, msg)
        cur = jnp.where(sel, -jnp.inf, cur)
    outcat = jnp.concatenate([feat_ref[...], msum * (1.0 / K), mmax], axis=1)
    obp = obp_ref[...]
    o = _mm(outcat, ow_ref[...]) + obp[0:1, :]
    o_ref[...] = o * (pg_ref[...] * BNS) + obp[1:2, :]


def _gravnet_call(s, bcol, brow, feat, h, p, i):
    ow = p[f'b{i}_oW'].T
    obp = jnp.stack([p[f'b{i}_ob'], p[f'b{i}_pb']])
    pg = p[f'b{i}_pg'][None, :]
    grid = (N // ROWT,)
    return pl.pallas_call(
        _gravnet_body2,
        grid=grid,
        in_specs=[
            pl.BlockSpec((ROWT, 16), lambda i: (i, 0)),
            pl.BlockSpec((ROWT, 1), lambda i: (i, 0)),
            pl.BlockSpec((ROWT, 64), lambda i: (i, 0)),
            pl.BlockSpec((N, 16), lambda i: (0, 0)),
            pl.BlockSpec((1, N), lambda i: (0, 0)),
            pl.BlockSpec((N, 64), lambda i: (0, 0)),
            pl.BlockSpec((192, 64), lambda i: (0, 0)),
            pl.BlockSpec((2, 64), lambda i: (0, 0)),
            pl.BlockSpec((1, 64), lambda i: (0, 0)),
        ],
        out_specs=pl.BlockSpec((ROWT, 64), lambda i: (i, 0)),
        out_shape=jax.ShapeDtypeStruct((N, 64), jnp.float32),
    )(s, bcol, feat, s, brow, h, ow, obp, pg)


def _head_body(c_ref, w0_ref, bb0_ref, g0_ref, w1_ref, bb1_ref, g1_ref,
               w2_ref, bb2_ref, g2_ref, wo_ref, bo_ref, o_ref):
    h = c_ref[...]
    for w_ref, bb_ref, g_ref in ((w0_ref, bb0_ref, g0_ref),
                                 (w1_ref, bb1_ref, g1_ref),
                                 (w2_ref, bb2_ref, g2_ref)):
        bb = bb_ref[...]
        h = jax.nn.relu(_mm(h, w_ref[...]) + bb[0:1, :])
        h = h * (g_ref[...] * BNS) + bb[1:2, :]
    o_ref[...] = jax.nn.sigmoid(_mm(h, wo_ref[...]) + bo_ref[...])


def _head_call(cat, p):
    args = [cat]
    for j in range(3):
        args.append(p[f'd{j}_W'].T)
        args.append(jnp.stack([p[f'd{j}_b'], p[f'd{j}_be']]))
        args.append(p[f'd{j}_g'][None, :])
    args.append(p['out_W'].T)
    args.append(p['out_b'][None, :])
    return pl.pallas_call(
        _head_body,
        out_shape=jax.ShapeDtypeStruct((N, 1), jnp.float32),
    )(*args)


def kernel(x, batch, params):
    b32 = batch.astype(jnp.int32)
    bcol = b32.reshape(N, 1)
    brow = b32.reshape(1, N)
    h = _global_exchange_call(x, bcol)
    feats = []
    for i in range(3):
        feat, s, hm = _dense_call(h, params, i)
        h = _gravnet_call(s, bcol, brow, feat, hm, params, i)
        feats.append(h)
    cat = jnp.concatenate(feats, axis=1)
    return _head_call(cat, params)


# speedup vs baseline: 11.5724x; 1.9335x over previous
import functools
import numpy as np
import jax
import jax.numpy as jnp
from jax import lax
from jax.experimental import pallas as pl
from jax.experimental.pallas import tpu as pltpu

N = 8192
NB = 8
K = 5
BNS = float(1.0 / np.sqrt(1.0 + 1e-5))

_NN = (((1,), (0,)), ((), ()))
_NT = (((1,), (1,)), ((), ()))
_TN = (((0,), (0,)), ((), ()))


def _mm(a, b):
    return lax.dot_general(a, b, _NN, preferred_element_type=jnp.float32)


def _mmt(a, b):
    return lax.dot_general(a, b, _NT, preferred_element_type=jnp.float32)


def _mtn(a, b):
    return lax.dot_general(a, b, _TN, preferred_element_type=jnp.float32)


def _ge_body(x_ref, b_ref, o_ref):
    x = x_ref[...]
    b = b_ref[...]
    oneh = (b == lax.broadcasted_iota(jnp.int32, (1, NB), 1)).astype(jnp.float32)
    ones_col = jnp.ones((N, 1), jnp.float32)
    cnt = _mtn(oneh, ones_col)
    ssum = _mtn(oneh, x)
    mean = ssum / jnp.maximum(cnt, 1.0)
    mns, mxs = [], []
    for bb in range(NB):
        m = b == bb
        mns.append(jnp.min(jnp.where(m, x, jnp.inf), axis=0, keepdims=True))
        mxs.append(jnp.max(jnp.where(m, x, -jnp.inf), axis=0, keepdims=True))
    mn = jnp.concatenate(mns, axis=0)
    mx = jnp.concatenate(mxs, axis=0)
    nonempty = cnt > 0.0
    mn = jnp.where(nonempty, mn, 0.0)
    mx = jnp.where(nonempty, mx, 0.0)
    mmm = jnp.concatenate([mean, mn, mx], axis=1)
    o_ref[...] = jnp.concatenate([_mm(oneh, mmm), x], axis=1)


def _global_exchange_call(x, bcol):
    return pl.pallas_call(
        _ge_body,
        out_shape=jax.ShapeDtypeStruct((N, 80), jnp.float32),
    )(x, bcol)


def _dense_body2(h_ref, w1_ref, bb1_ref, g1_ref, w2_ref, bb2_ref, g2_ref,
                 w3_ref, b3_ref, sw_ref, sb_ref, hw_ref, hb_ref,
                 feat_ref, s_ref, hm_ref):
    h = h_ref[...]
    bb1 = bb1_ref[...]
    x1 = jax.nn.relu(_mm(h, w1_ref[...]) + bb1[0:1, :])
    x1 = x1 * (g1_ref[...] * BNS) + bb1[1:2, :]
    bb2 = bb2_ref[...]
    x2 = jax.nn.relu(_mm(x1, w2_ref[...]) + bb2[0:1, :])
    x2 = x2 * (g2_ref[...] * BNS) + bb2[1:2, :]
    feat = _mm(x2, w3_ref[...]) + b3_ref[...]
    feat_ref[...] = feat
    s_ref[...] = _mm(feat, sw_ref[...]) + sb_ref[...]
    hm_ref[...] = _mm(feat, hw_ref[...]) + hb_ref[...]


def _dense_call(h, p, i):
    w1 = p[f'b{i}_W1'].T
    bb1 = jnp.stack([p[f'b{i}_b1'], p[f'b{i}_be1']])
    g1 = p[f'b{i}_g1'][None, :]
    w2 = p[f'b{i}_W2'].T
    bb2 = jnp.stack([p[f'b{i}_b2'], p[f'b{i}_be2']])
    g2 = p[f'b{i}_g2'][None, :]
    w3 = p[f'b{i}_W3'].T
    b3 = p[f'b{i}_b3'][None, :]
    sw = p[f'b{i}_sW'].T
    sb = p[f'b{i}_sb'][None, :]
    hw = p[f'b{i}_hW'].T
    hb = p[f'b{i}_hb'][None, :]
    return pl.pallas_call(
        _dense_body2,
        out_shape=[
            jax.ShapeDtypeStruct((N, 64), jnp.float32),
            jax.ShapeDtypeStruct((N, 16), jnp.float32),
            jax.ShapeDtypeStruct((N, 64), jnp.float32),
        ],
    )(h, w1, bb1, g1, w2, bb2, g2, w3, b3, sw, sb, hw, hb)


ROWT = 256
NEG_BIG = -1e9


def _gravnet_body2(sr_ref, br_ref, feat_ref, s_ref, ball_ref, h_ref,
                   ow_ref, obp_ref, pg_ref, o_ref):
    s_r = sr_ref[...]
    s_a = s_ref[...]
    sq_r = jnp.sum(s_r * s_r, axis=1, keepdims=True)
    sq_a = jnp.sum(s_a * s_a, axis=1, keepdims=True)
    aug_r = jnp.concatenate([s_r * -2.0, jnp.ones((ROWT, 1), jnp.float32)],
                            axis=1)
    aug_a = jnp.concatenate([s_a, sq_a], axis=1)
    d2 = sq_r + _mmt(aug_r, aug_a)
    d2 = jnp.maximum(d2, 0.0)
    same = br_ref[...] == ball_ref[...]
    cur = jnp.where(same, -d2, NEG_BIG)
    cols = lax.broadcasted_iota(jnp.int32, (ROWT, N), 1)
    h_a = h_ref[...]
    msum = jnp.zeros((ROWT, 64), jnp.float32)
    mmax = jnp.full((ROWT, 64), -jnp.inf, jnp.float32)
    for _ in range(K):
        mval = jnp.max(cur, axis=1, keepdims=True)
        eq = cur == mval
        amax = jnp.min(jnp.where(eq, cols, jnp.int32(1 << 30)),
                       axis=1, keepdims=True)
        sel = cols == amax
        oneh = sel.astype(jnp.float32)
        g = _mm(oneh, h_a)
        msg = g * jnp.exp(10.0 * mval)
        msum = msum + msg
        mmax = jnp.maximum(mmax, msg)
        cur = jnp.where(sel, -jnp.inf, cur)
    outcat = jnp.concatenate([feat_ref[...], msum * (1.0 / K), mmax], axis=1)
    obp = obp_ref[...]
    o = _mm(outcat, ow_ref[...]) + obp[0:1, :]
    o_ref[...] = o * (pg_ref[...] * BNS) + obp[1:2, :]


CW = 1024


def _gravnet_win_body(clo_ref, chi_ref, sr_ref, br_ref, feat_ref, s_ref,
                      ball_ref, h_ref, ow_ref, obp_ref, pg_ref, o_ref):
    t = pl.program_id(0)
    c0 = clo_ref[t]
    c1 = chi_ref[t]
    s_r = sr_ref[...]
    sq_r = jnp.sum(s_r * s_r, axis=1, keepdims=True)
    aug_r = jnp.concatenate([s_r * -2.0, jnp.ones((ROWT, 1), jnp.float32)],
                            axis=1)
    br = br_ref[...]
    liota = lax.broadcasted_iota(jnp.int32, (ROWT, CW), 1)

    def chunk_negd(j):
        base = j * CW
        s_c = s_ref[pl.ds(base, CW), :]
        sq_c = jnp.sum(s_c * s_c, axis=1, keepdims=True)
        aug_c = jnp.concatenate([s_c, sq_c], axis=1)
        d2 = sq_r + _mmt(aug_r, aug_c)
        d2 = jnp.maximum(d2, 0.0)
        bc = ball_ref[:, pl.ds(base, CW)]
        same = br == bc
        return jnp.where(same, -d2, NEG_BIG)

    def phase1(j, carry):
        bval, bidx = carry
        negd = chunk_negd(j)
        gcols = liota + j * CW
        cval = jnp.concatenate([bval, negd], axis=1)
        cidx = jnp.concatenate([bidx, gcols], axis=1)
        nv, ni = [], []
        for _ in range(K):
            m = jnp.max(cval, axis=1, keepdims=True)
            eq = cval == m
            gidx = jnp.min(jnp.where(eq, cidx, jnp.int32(1 << 30)),
                           axis=1, keepdims=True)
            cval = jnp.where(eq & (cidx == gidx), -jnp.inf, cval)
            nv.append(m)
            ni.append(gidx)
        return (jnp.concatenate(nv, axis=1), jnp.concatenate(ni, axis=1))

    bval0 = jnp.full((ROWT, K), -jnp.inf, jnp.float32)
    bidx0 = jnp.full((ROWT, K), jnp.int32(1 << 30))
    bval, bidx = lax.fori_loop(c0, c1, phase1, (bval0, bidx0))

    def phase2(j, carry):
        base = j * CW
        h_c = h_ref[pl.ds(base, CW), :]
        gcols = liota + base
        out = []
        for k in range(K):
            oneh = (gcols == bidx[:, k:k + 1]).astype(jnp.float32)
            out.append(carry[k] + _mm(oneh, h_c))
        return tuple(out)

    g0 = tuple(jnp.zeros((ROWT, 64), jnp.float32) for _ in range(K))
    gs = lax.fori_loop(c0, c1, phase2, g0)

    msum = jnp.zeros((ROWT, 64), jnp.float32)
    mmax = jnp.full((ROWT, 64), -jnp.inf, jnp.float32)
    for k in range(K):
        msg = gs[k] * jnp.exp(10.0 * bval[:, k:k + 1])
        msum = msum + msg
        mmax = jnp.maximum(mmax, msg)
    outcat = jnp.concatenate([feat_ref[...], msum * (1.0 / K), mmax], axis=1)
    obp = obp_ref[...]
    o = _mm(outcat, ow_ref[...]) + obp[0:1, :]
    o_ref[...] = o * (pg_ref[...] * BNS) + obp[1:2, :]


def _gravnet_win_call(s, bcol, brow, feat, h, p, i, clo, chi):
    ow = p[f'b{i}_oW'].T
    obp = jnp.stack([p[f'b{i}_ob'], p[f'b{i}_pb']])
    pg = p[f'b{i}_pg'][None, :]
    grid = (N // ROWT,)
    return pl.pallas_call(
        _gravnet_win_body,
        grid_spec=pltpu.PrefetchScalarGridSpec(
            num_scalar_prefetch=2,
            grid=grid,
            in_specs=[
                pl.BlockSpec((ROWT, 16), lambda i, *_: (i, 0)),
                pl.BlockSpec((ROWT, 1), lambda i, *_: (i, 0)),
                pl.BlockSpec((ROWT, 64), lambda i, *_: (i, 0)),
                pl.BlockSpec((N, 16), lambda i, *_: (0, 0)),
                pl.BlockSpec((1, N), lambda i, *_: (0, 0)),
                pl.BlockSpec((N, 64), lambda i, *_: (0, 0)),
                pl.BlockSpec((192, 64), lambda i, *_: (0, 0)),
                pl.BlockSpec((2, 64), lambda i, *_: (0, 0)),
                pl.BlockSpec((1, 64), lambda i, *_: (0, 0)),
            ],
            out_specs=pl.BlockSpec((ROWT, 64), lambda i, *_: (i, 0)),
        ),
        out_shape=jax.ShapeDtypeStruct((N, 64), jnp.float32),
    )(clo, chi, s, bcol, feat, s, brow, h, ow, obp, pg)


def _gravnet_call(s, bcol, brow, feat, h, p, i):
    ow = p[f'b{i}_oW'].T
    obp = jnp.stack([p[f'b{i}_ob'], p[f'b{i}_pb']])
    pg = p[f'b{i}_pg'][None, :]
    grid = (N // ROWT,)
    return pl.pallas_call(
        _gravnet_body2,
        grid=grid,
        in_specs=[
            pl.BlockSpec((ROWT, 16), lambda i: (i, 0)),
            pl.BlockSpec((ROWT, 1), lambda i: (i, 0)),
            pl.BlockSpec((ROWT, 64), lambda i: (i, 0)),
            pl.BlockSpec((N, 16), lambda i: (0, 0)),
            pl.BlockSpec((1, N), lambda i: (0, 0)),
            pl.BlockSpec((N, 64), lambda i: (0, 0)),
            pl.BlockSpec((192, 64), lambda i: (0, 0)),
            pl.BlockSpec((2, 64), lambda i: (0, 0)),
            pl.BlockSpec((1, 64), lambda i: (0, 0)),
        ],
        out_specs=pl.BlockSpec((ROWT, 64), lambda i: (i, 0)),
        out_shape=jax.ShapeDtypeStruct((N, 64), jnp.float32),
    )(s, bcol, feat, s, brow, h, ow, obp, pg)


def _head_body(c_ref, w0_ref, bb0_ref, g0_ref, w1_ref, bb1_ref, g1_ref,
               w2_ref, bb2_ref, g2_ref, wo_ref, bo_ref, o_ref):
    h = c_ref[...]
    for w_ref, bb_ref, g_ref in ((w0_ref, bb0_ref, g0_ref),
                                 (w1_ref, bb1_ref, g1_ref),
                                 (w2_ref, bb2_ref, g2_ref)):
        bb = bb_ref[...]
        h = jax.nn.relu(_mm(h, w_ref[...]) + bb[0:1, :])
        h = h * (g_ref[...] * BNS) + bb[1:2, :]
    o_ref[...] = jax.nn.sigmoid(_mm(h, wo_ref[...]) + bo_ref[...])


def _head_call(cat, p):
    args = [cat]
    for j in range(3):
        args.append(p[f'd{j}_W'].T)
        args.append(jnp.stack([p[f'd{j}_b'], p[f'd{j}_be']]))
        args.append(p[f'd{j}_g'][None, :])
    args.append(p['out_W'].T)
    args.append(p['out_b'][None, :])
    return pl.pallas_call(
        _head_body,
        out_shape=jax.ShapeDtypeStruct((N, 1), jnp.float32),
    )(*args)


def kernel(x, batch, params):
    b32 = batch.astype(jnp.int32)
    bcol = b32.reshape(N, 1)
    brow = b32.reshape(1, N)
    seg_start = jnp.searchsorted(b32, jnp.arange(NB, dtype=jnp.int32),
                                 side='left').astype(jnp.int32)
    seg_end = jnp.searchsorted(b32, jnp.arange(NB, dtype=jnp.int32),
                               side='right').astype(jnp.int32)
    bt0 = b32[::ROWT]
    bt1 = b32[ROWT - 1::ROWT]
    clo = (seg_start[bt0] // CW).astype(jnp.int32)
    chi = ((seg_end[bt1] + CW - 1) // CW).astype(jnp.int32)
    h = _global_exchange_call(x, bcol)
    feats = []
    for i in range(3):
        feat, s, hm = _dense_call(h, params, i)
        h = _gravnet_win_call(s, bcol, brow, feat, hm, params, i, clo, chi)
        feats.append(h)
    cat = jnp.concatenate(feats, axis=1)
    return _head_call(cat, params)
